# baseline (device time: 26058 ns/iter reference)
import jax
import jax.numpy as jnp
from jax import lax
from jax.experimental import pallas as pl
from jax.experimental.pallas import tpu as pltpu

N_DEV = 8
E_PER = 2
N_EXP = N_DEV * E_PER
CAP = 204


def kernel(x, router_W, route_idx, expert_W):
    del router_W
    m, d = x.shape
    _, _, h = expert_W.shape

    def body(x_ref, route_ref, w_ref, out_ref,
             wg_ref, sideg_ref, send_w, recv_w, send_c, recv_c):
        my = lax.axis_index("i")

        route = route_ref[...]
        e_iota = lax.broadcasted_iota(jnp.int32, (m, N_EXP), 1)
        oh = (route == e_iota).astype(jnp.float32)

        w_val = w_ref[...]
        scale = jnp.max(jnp.abs(w_val)).reshape(1, 1)
        counts = jnp.sum(oh, axis=0, keepdims=True)
        sideg_ref[my] = jnp.concatenate(
            [counts, jnp.broadcast_to(scale, (1, N_EXP))], axis=1
        )

        wg_ref[my] = jnp.round(w_val * (127.0 / scale)).astype(jnp.int8)

        barrier = pltpu.get_barrier_semaphore()
        for k in range(1, N_DEV):
            pl.semaphore_signal(
                barrier, inc=1,
                device_id=(lax.rem(my + k, N_DEV),),
                device_id_type=pl.DeviceIdType.MESH,
            )
        pl.semaphore_wait(barrier, N_DEV - 1)

        sends = []
        for k in range(1, N_DEV):
            dst = lax.rem(my + k, N_DEV)
            rw = pltpu.make_async_remote_copy(
                src_ref=wg_ref.at[my],
                dst_ref=wg_ref.at[my],
                send_sem=send_w.at[k - 1],
                recv_sem=recv_w.at[k - 1],
                device_id=(dst,),
                device_id_type=pl.DeviceIdType.MESH,
            )
            rc = pltpu.make_async_remote_copy(
                src_ref=sideg_ref.at[my],
                dst_ref=sideg_ref.at[my],
                send_sem=send_c.at[k - 1],
                recv_sem=recv_c.at[k - 1],
                device_id=(dst,),
                device_id_type=pl.DeviceIdType.MESH,
            )
            rw.start()
            rc.start()
            sends.append((rw, rc))

        x_val = x_ref[...].astype(jnp.bfloat16)

        def compute_chunk(p, w8_chunk, scale_p, acc):
            m0 = (route == 2 * p).astype(jnp.bfloat16)
            m1 = (route == 2 * p + 1).astype(jnp.bfloat16)
            xm = jnp.concatenate([x_val * m0, x_val * m1], axis=1)
            wv = (
                w8_chunk.reshape(E_PER * d, h).astype(jnp.float32)
                * (scale_p * (1.0 / 127.0))
            ).astype(jnp.bfloat16)
            return acc + jnp.dot(xm, wv, preferred_element_type=jnp.float32)

        acc = compute_chunk(
            my, wg_ref[my], scale, jnp.zeros((m, h), jnp.float32)
        )

        for k in range(1, N_DEV):
            p = lax.rem(my - k + N_DEV, N_DEV)
            recv_w_desc = pltpu.make_async_remote_copy(
                src_ref=wg_ref.at[p],
                dst_ref=wg_ref.at[p],
                send_sem=send_w.at[k - 1],
                recv_sem=recv_w.at[k - 1],
                device_id=(my,),
                device_id_type=pl.DeviceIdType.MESH,
            )
            recv_w_desc.wait_recv()
            recv_c_desc = pltpu.make_async_remote_copy(
                src_ref=sideg_ref.at[p],
                dst_ref=sideg_ref.at[p],
                send_sem=send_c.at[k - 1],
                recv_sem=recv_c.at[k - 1],
                device_id=(my,),
                device_id_type=pl.DeviceIdType.MESH,
            )
            recv_c_desc.wait_recv()
            scale_p = sideg_ref[p, :, N_EXP:N_EXP + 1]
            acc = compute_chunk(p, wg_ref[p], scale_p, acc)

        row = lax.broadcasted_iota(jnp.int32, (m, m), 0)
        col = lax.broadcasted_iota(jnp.int32, (m, m), 1)
        tri = (col < row).astype(jnp.float32)
        pos = jnp.dot(tri, oh, preferred_element_type=jnp.float32)

        allcounts = sideg_ref[:, :, :N_EXP]
        dev_iota = lax.broadcasted_iota(jnp.int32, (N_DEV, 1, N_EXP), 0)
        prior = (dev_iota < my).astype(jnp.float32)
        base = jnp.sum(allcounts * prior, axis=0)

        keep = jnp.sum(
            oh * (pos + base < CAP).astype(jnp.float32), axis=1, keepdims=True
        )
        out_ref[...] = acc * keep

        for rw, rc in sends:
            rw.wait_send()
            rc.wait_send()

    return pl.pallas_call(
        body,
        out_shape=jax.ShapeDtypeStruct((m, h), jnp.float32),
        in_specs=[
            pl.BlockSpec(memory_space=pltpu.VMEM),
            pl.BlockSpec(memory_space=pltpu.VMEM),
            pl.BlockSpec(memory_space=pltpu.VMEM),
        ],
        out_specs=pl.BlockSpec(memory_space=pltpu.VMEM),
        scratch_shapes=[
            pltpu.VMEM((N_DEV, E_PER, d, h), jnp.int8),
            pltpu.VMEM((N_DEV, 1, 2 * N_EXP), jnp.float32),
            pltpu.SemaphoreType.DMA((N_DEV - 1,)),
            pltpu.SemaphoreType.DMA((N_DEV - 1,)),
            pltpu.SemaphoreType.DMA((N_DEV - 1,)),
            pltpu.SemaphoreType.DMA((N_DEV - 1,)),
        ],
        compiler_params=pltpu.CompilerParams(collective_id=0),
    )(x, route_idx, expert_W)
